# baseline (speedup 1.0000x reference)
import functools

import jax
import jax.numpy as jnp
from jax import lax
from jax.experimental import pallas as pl
from jax.experimental.pallas import tpu as pltpu
from jax.experimental.pallas import tpu_sc as plsc

N = 10000
E = 320000
N_PAD = 10240
NS = 16
LANES = 16
C = 128
N_CHUNKS = 158
EPT = N_CHUNKS * C
E_PAD = EPT * NS
ROWS_PT = N_PAD // NS
RB = 128
N_RB = ROWS_PT // RB

f32 = jnp.float32


def _spmm_call(dh, first):
    mesh = plsc.VectorSubcoreMesh(core_axis_name="c", subcore_axis_name="s")
    out_type = [jax.ShapeDtypeStruct((2 * N_PAD, dh), f32)]
    if first:
        out_type.append(jax.ShapeDtypeStruct((N_PAD, LANES), f32))
    NSLOT = 4
    scratch = [
        pltpu.VMEM((N_CHUNKS, C), jnp.int32),
        pltpu.VMEM((N_CHUNKS, C), jnp.int32),
        pltpu.VMEM((NSLOT, C, dh), f32),
        pltpu.VMEM((2, RB, LANES), f32),
        pltpu.VMEM_SHARED((N_PAD, dh), f32),
    ]
    scratch += [pltpu.SemaphoreType.DMA] * (2 * NSLOT)
    if first:
        scratch.append(pltpu.VMEM((C, LANES), f32))
        scratch.append(pltpu.VMEM_SHARED((N_PAD, LANES), f32))
        scratch += [pltpu.SemaphoreType.DMA] * NSLOT

    def body(src_hbm, dst_hbm, x_hbm, *rest):
        if first:
            (y_hbm, inv_hbm, srcb, dstb, rows, ivw, acc, *sems) = rest
            gsem, ssem = sems[:NSLOT], sems[NSLOT:2 * NSLOT]
            ones_v, deg_sh = sems[2 * NSLOT], sems[2 * NSLOT + 1]
            dsem = sems[2 * NSLOT + 2:]
        else:
            (inv_in, y_hbm, srcb, dstb, rows, ivw, acc, *sems) = rest
            gsem, ssem = sems[:NSLOT], sems[NSLOT:2 * NSLOT]
        s_idx = lax.axis_index("s")
        c_idx = lax.axis_index("c")

        def zrow(r, _):
            for j in range(dh // LANES):
                rows[0, r, pl.ds(j * LANES, LANES)] = jnp.zeros((LANES,), f32)
            if first:
                ones_v[r, :] = jnp.zeros((LANES,), f32)
            return 0

        lax.fori_loop(0, RB, zrow, 0)
        zero_descs = []
        for i in range(N_RB):
            row0 = s_idx * ROWS_PT + i * RB
            zero_descs.append(
                pltpu.async_copy(rows.at[0], acc.at[pl.ds(row0, RB)], ssem[0]))
            if first:
                zero_descs.append(
                    pltpu.async_copy(ones_v, deg_sh.at[pl.ds(row0, RB)],
                                     ssem[1]))

        pltpu.sync_copy(src_hbm.at[s_idx], srcb)
        pltpu.sync_copy(dst_hbm.at[s_idx], dstb)

        @pl.when(c_idx != 0)
        def _():
            def offrow(g, _):
                for j in range(C // LANES):
                    sl = pl.ds(j * LANES, LANES)
                    srcb[g, sl] = srcb[g, sl] + N_PAD
                return 0

            lax.fori_loop(0, N_CHUNKS, offrow, 0)

        for d in zero_descs:
            d.wait()
        if first:
            def orow(r, _):
                ones_v[r, :] = jnp.ones((LANES,), f32)
                return 0

            lax.fori_loop(0, RB, orow, 0)
        plsc.subcore_barrier()

        def start_gather(g, b):
            pltpu.async_copy(x_hbm.at[srcb.at[g]], rows.at[b], gsem[b])

        def wait_gather(b):
            pltpu.make_async_copy(x_hbm.at[srcb.at[0]], rows.at[b],
                                  gsem[b]).wait()

        def start_scatter(g, b):
            pltpu.async_copy(rows.at[b], acc.at[dstb.at[g]], ssem[b], add=True)
            if first:
                pltpu.async_copy(ones_v, deg_sh.at[dstb.at[g]], dsem[b],
                                 add=True)

        def wait_scatter(b):
            pltpu.make_async_copy(rows.at[b], acc.at[dstb.at[0]],
                                  ssem[b]).wait()
            if first:
                pltpu.make_async_copy(ones_v, deg_sh.at[dstb.at[0]],
                                      dsem[b]).wait()

        start_gather(0, 0)
        start_gather(1, 1)
        for g in range(2, NSLOT + 2):
            b, b2 = g % NSLOT, (g - 2) % NSLOT
            if g - NSLOT >= 0:
                wait_scatter(b)
            start_gather(g, b)
            wait_gather(b2)
            start_scatter(g - 2, b2)

        steady0 = NSLOT + 2
        n_steady = ((N_CHUNKS - steady0) // NSLOT) * NSLOT

        def pipe(t, _):
            for u in range(NSLOT):
                g = steady0 + NSLOT * t + u
                b, b2 = (steady0 + u) % NSLOT, (steady0 + u - 2) % NSLOT
                wait_scatter(b)
                start_gather(g, b)
                wait_gather(b2)
                start_scatter(g - 2, b2)
            return 0

        lax.fori_loop(0, n_steady // NSLOT, pipe, 0)
        for g in range(steady0 + n_steady, N_CHUNKS):
            b, b2 = g % NSLOT, (g - 2) % NSLOT
            wait_scatter(b)
            start_gather(g, b)
            wait_gather(b2)
            start_scatter(g - 2, b2)
        for g in (N_CHUNKS - 2, N_CHUNKS - 1):
            b = g % NSLOT
            wait_gather(b)
            start_scatter(g, b)
        for b in range(NSLOT):
            wait_scatter(b)
        plsc.subcore_barrier()

        def rblk(i):
            return s_idx * ROWS_PT + i * RB

        out_descs = {}
        for i in range(N_RB):
            b = i % 2
            if i >= 2:
                out_descs.pop(i - 2).wait()
            pltpu.sync_copy(acc.at[pl.ds(rblk(i), RB)], rows.at[b])
            if first:
                pltpu.sync_copy(deg_sh.at[pl.ds(rblk(i), RB)], ivw.at[b])
            else:
                pltpu.sync_copy(inv_in.at[pl.ds(rblk(i), RB)], ivw.at[b])

            def srow(r, _):
                if first:
                    ivv = 1.0 / jnp.maximum(ivw[b, r, :], 1.0)
                    ivw[b, r, :] = ivv
                else:
                    ivv = ivw[b, r, :]
                for j in range(dh // LANES):
                    sl = pl.ds(j * LANES, LANES)
                    rows[b, r, sl] = rows[b, r, sl] * ivv
                return 0

            lax.fori_loop(0, RB, srow, 0)
            out_descs[i] = pltpu.async_copy(
                rows.at[b], y_hbm.at[pl.ds(c_idx * N_PAD + rblk(i), RB)],
                ssem[b])
            if first:
                @pl.when(c_idx == 0)
                def _():
                    pltpu.sync_copy(ivw.at[b], inv_hbm.at[pl.ds(rblk(i), RB)])
        for i in sorted(out_descs):
            out_descs[i].wait()

    return pl.kernel(body, out_type=tuple(out_type) if first else out_type[0],
                     mesh=mesh, scratch_types=scratch,
                     compiler_params=pltpu.CompilerParams(
                         use_tc_tiling_on_sc=False))


TC_R = 2048


def _partial_call(dh_in, d_out):
    xs_spec = pl.BlockSpec((2, TC_R, dh_in), lambda i: (0, i, 0))
    w_spec = pl.BlockSpec((2 * dh_in, d_out), lambda i: (0, 0))
    b_spec = pl.BlockSpec((1, d_out), lambda i: (0, 0))

    def body(x0r, s1r, war, wbr, br, out_r):
        out_r[...] = (jnp.dot(x0r[0], war[:dh_in])
                      + jnp.dot(x0r[1], war[dh_in:])
                      + jnp.dot(s1r[0], wbr[:dh_in])
                      + jnp.dot(s1r[1], wbr[dh_in:]) + br[0])

    return pl.pallas_call(
        body, out_shape=jax.ShapeDtypeStruct((N_PAD, d_out), f32),
        grid=(N_PAD // TC_R,),
        in_specs=[xs_spec, xs_spec, w_spec, w_spec, b_spec],
        out_specs=pl.BlockSpec((TC_R, d_out), lambda i: (i, 0)))


def _combine_call(dh_in, d_out, dh_out):
    p_spec = pl.BlockSpec((TC_R, d_out), lambda i: (i, 0))
    xs_spec = pl.BlockSpec((2, TC_R, dh_in), lambda i: (0, i, 0))
    w_spec = pl.BlockSpec((2 * dh_in, d_out), lambda i: (0, 0))
    if dh_out:
        out_shape = jax.ShapeDtypeStruct((2, N_PAD, dh_out), f32)
        out_spec = pl.BlockSpec((2, TC_R, dh_out), lambda i: (0, i, 0))
    else:
        out_shape = jax.ShapeDtypeStruct((N_PAD, d_out), f32)
        out_spec = pl.BlockSpec((TC_R, d_out), lambda i: (i, 0))

    def body(pr, s2r, wcr, out_r):
        h = (pr[...] + jnp.dot(s2r[0], wcr[:dh_in])
             + jnp.dot(s2r[1], wcr[dh_in:]))
        h = jnp.maximum(h, 0.0)
        if dh_out:
            out_r[0] = h[:, :dh_out]
            rw = d_out - dh_out
            right = h[:, dh_out:d_out]
            if rw < dh_out:
                right = jnp.concatenate(
                    [right, jnp.zeros((TC_R, dh_out - rw), f32)], axis=1)
            out_r[1] = right
        else:
            out_r[...] = h

    return pl.pallas_call(
        body, out_shape=out_shape, grid=(N_PAD // TC_R,),
        in_specs=[p_spec, xs_spec, w_spec],
        out_specs=out_spec)


def _final_call():
    out_shape = jax.ShapeDtypeStruct((N_PAD, 10), f32)

    def body(h_r, w2_r, b2_r, w3_r, b3_r, out_r):
        t = jnp.dot(h_r[...], w2_r[...]) + b2_r[0]
        t = jnp.dot(t, w3_r[...]) + b3_r[0]
        t = t - jnp.max(t, axis=-1, keepdims=True)
        e = jnp.exp(t)
        out_r[...] = e / jnp.sum(e, axis=-1, keepdims=True)

    return pl.pallas_call(
        body, out_shape=out_shape, grid=(N_PAD // TC_R,),
        in_specs=[pl.BlockSpec((TC_R, 256), lambda i: (i, 0)),
                  pl.BlockSpec((256, 300), lambda i: (0, 0)),
                  pl.BlockSpec((1, 300), lambda i: (0, 0)),
                  pl.BlockSpec((300, 10), lambda i: (0, 0)),
                  pl.BlockSpec((1, 10), lambda i: (0, 0))],
        out_specs=pl.BlockSpec((TC_R, 10), lambda i: (i, 0)))


def _prep_w(W, din_real, din_p):
    W0, W1_, W2 = W[0::3], W[1::3], W[2::3]
    mats = (W0 - W2, W1_, 2.0 * W2)
    if din_p != din_real:
        mats = tuple(jnp.pad(m, ((0, din_p - din_real), (0, 0))) for m in mats)
    return mats


def kernel(x, edge_index, W1, b1, W2, b2, W3, b3, W4, b4, Wl2, bl2, Wl3, bl3):
    pad_e = E_PAD - E
    src = jnp.concatenate(
        [edge_index[0], jnp.full((pad_e,), N, jnp.int32)]
    ).reshape(NS, N_CHUNKS, C)
    dst = jnp.concatenate(
        [edge_index[1], jnp.full((pad_e,), N, jnp.int32)]
    ).reshape(NS, N_CHUNKS, C)
    xp = jnp.pad(x, ((0, N_PAD - N), (0, 0)))
    x0s = jnp.concatenate([xp[:, :64], xp[:, 64:]], axis=0)

    dims = [(128, 128, 48, 32), (48, 64, 96, 48),
            (96, 96, 128, 64), (128, 128, 256, 0)]
    Ws = [W1, W2, W3, W4]
    bs = [b1, b2, b3, b4]

    inv = None
    for li, (din, dinp, dout, dhout) in enumerate(dims):
        dh = dinp // 2
        x3 = x0s.reshape(2, N_PAD, dh)
        if li == 0:
            s1, inv = _spmm_call(dh, True)(src, dst, x0s)
        else:
            s1 = _spmm_call(dh, False)(src, dst, x0s, inv)
        s2 = _spmm_call(dh, False)(src, dst, s1, inv)
        wa, wb, wc = _prep_w(Ws[li], din, dinp)
        part = _partial_call(dh, dout)(
            x3, s1.reshape(2, N_PAD, dh), wa, wb, bs[li].reshape(1, dout))
        out = _combine_call(dh, dout, dhout)(
            part, s2.reshape(2, N_PAD, dh), wc)
        x0s = out.reshape(2 * N_PAD, dhout) if dhout else out

    o = _final_call()(x0s, Wl2, bl2.reshape(1, 300), Wl3, bl3.reshape(1, 10))
    return o[:N]


# speedup vs baseline: 1.0064x; 1.0064x over previous
import functools

import jax
import jax.numpy as jnp
from jax import lax
from jax.experimental import pallas as pl
from jax.experimental.pallas import tpu as pltpu
from jax.experimental.pallas import tpu_sc as plsc

N = 10000
E = 320000
N_PAD = 10240
NS = 16
LANES = 16
C = 128
N_CHUNKS = 158
EPT = N_CHUNKS * C
E_PAD = EPT * NS
ROWS_PT = N_PAD // NS
RB = 128
N_RB = ROWS_PT // RB

f32 = jnp.float32


def _spmm_call(dh, first):
    mesh = plsc.VectorSubcoreMesh(core_axis_name="c", subcore_axis_name="s")
    out_type = [jax.ShapeDtypeStruct((2 * N_PAD, dh), f32)]
    if first:
        out_type.append(jax.ShapeDtypeStruct((N_PAD, LANES), f32))
    NSLOT = 4
    scratch = [
        pltpu.VMEM((N_CHUNKS, C), jnp.int32),
        pltpu.VMEM((N_CHUNKS, C), jnp.int32),
        pltpu.VMEM((NSLOT, C, dh), f32),
        pltpu.VMEM((2, RB, LANES), f32),
        pltpu.VMEM_SHARED((N_PAD, dh), f32),
    ]
    scratch += [pltpu.SemaphoreType.DMA] * (2 * NSLOT)
    if first:
        scratch.append(pltpu.VMEM((C, LANES), f32))
        scratch.append(pltpu.VMEM_SHARED((N_PAD, LANES), f32))
        scratch += [pltpu.SemaphoreType.DMA] * NSLOT

    def body(src_hbm, dst_hbm, x_hbm, *rest):
        if first:
            (y_hbm, inv_hbm, srcb, dstb, rows, ivw, acc, *sems) = rest
            gsem, ssem = sems[:NSLOT], sems[NSLOT:2 * NSLOT]
            ones_v, deg_sh = sems[2 * NSLOT], sems[2 * NSLOT + 1]
            dsem = sems[2 * NSLOT + 2:]
        else:
            (inv_in, y_hbm, srcb, dstb, rows, ivw, acc, *sems) = rest
            gsem, ssem = sems[:NSLOT], sems[NSLOT:2 * NSLOT]
        s_idx = lax.axis_index("s")
        c_idx = lax.axis_index("c")

        def zrow(r, _):
            for j in range(dh // LANES):
                rows[0, r, pl.ds(j * LANES, LANES)] = jnp.zeros((LANES,), f32)
            if first:
                ones_v[r, :] = jnp.zeros((LANES,), f32)
            return 0

        lax.fori_loop(0, RB, zrow, 0)
        zero_descs = []
        for i in range(N_RB):
            row0 = s_idx * ROWS_PT + i * RB
            zero_descs.append(
                pltpu.async_copy(rows.at[0], acc.at[pl.ds(row0, RB)], ssem[0]))
            if first:
                zero_descs.append(
                    pltpu.async_copy(ones_v, deg_sh.at[pl.ds(row0, RB)],
                                     ssem[1]))

        pltpu.sync_copy(src_hbm.at[s_idx], srcb)
        pltpu.sync_copy(dst_hbm.at[s_idx], dstb)

        @pl.when(c_idx != 0)
        def _():
            def offrow(g, _):
                for j in range(C // LANES):
                    sl = pl.ds(j * LANES, LANES)
                    srcb[g, sl] = srcb[g, sl] + N_PAD
                return 0

            lax.fori_loop(0, N_CHUNKS, offrow, 0)

        for d in zero_descs:
            d.wait()
        if first:
            def orow(r, _):
                ones_v[r, :] = jnp.ones((LANES,), f32)
                return 0

            lax.fori_loop(0, RB, orow, 0)
        plsc.subcore_barrier()

        def start_gather(g, b):
            pltpu.async_copy(x_hbm.at[srcb.at[g]], rows.at[b], gsem[b])

        def wait_gather(b):
            pltpu.make_async_copy(x_hbm.at[srcb.at[0]], rows.at[b],
                                  gsem[b]).wait()

        def start_scatter(g, b):
            pltpu.async_copy(rows.at[b], acc.at[dstb.at[g]], ssem[b], add=True)
            if first:
                pltpu.async_copy(ones_v, deg_sh.at[dstb.at[g]], dsem[b],
                                 add=True)

        def wait_scatter(b):
            pltpu.make_async_copy(rows.at[b], acc.at[dstb.at[0]],
                                  ssem[b]).wait()
            if first:
                pltpu.make_async_copy(ones_v, deg_sh.at[dstb.at[0]],
                                      dsem[b]).wait()

        start_gather(0, 0)
        start_gather(1, 1)
        for g in range(2, NSLOT + 2):
            b, b2 = g % NSLOT, (g - 2) % NSLOT
            if g - NSLOT >= 0:
                wait_scatter(b)
            start_gather(g, b)
            wait_gather(b2)
            start_scatter(g - 2, b2)

        steady0 = NSLOT + 2
        n_steady = ((N_CHUNKS - steady0) // NSLOT) * NSLOT

        def pipe(t, _):
            for u in range(NSLOT):
                g = steady0 + NSLOT * t + u
                b, b2 = (steady0 + u) % NSLOT, (steady0 + u - 2) % NSLOT
                wait_scatter(b)
                start_gather(g, b)
                wait_gather(b2)
                start_scatter(g - 2, b2)
            return 0

        lax.fori_loop(0, n_steady // NSLOT, pipe, 0)
        for g in range(steady0 + n_steady, N_CHUNKS):
            b, b2 = g % NSLOT, (g - 2) % NSLOT
            wait_scatter(b)
            start_gather(g, b)
            wait_gather(b2)
            start_scatter(g - 2, b2)
        for g in (N_CHUNKS - 2, N_CHUNKS - 1):
            b = g % NSLOT
            wait_gather(b)
            start_scatter(g, b)
        for b in range(NSLOT):
            wait_scatter(b)
        plsc.subcore_barrier()

        def rblk(i):
            return s_idx * ROWS_PT + i * RB

        out_descs = {}
        for i in range(N_RB):
            b = i % 2
            if i >= 2:
                out_descs.pop(i - 2).wait()
            pltpu.sync_copy(acc.at[pl.ds(rblk(i), RB)], rows.at[b])
            if first:
                pltpu.sync_copy(deg_sh.at[pl.ds(rblk(i), RB)], ivw.at[b])
            else:
                pltpu.sync_copy(inv_in.at[pl.ds(rblk(i), RB)], ivw.at[b])

            def srow(r, _):
                if first:
                    ivv = 1.0 / jnp.maximum(ivw[b, r, :], 1.0)
                    ivw[b, r, :] = ivv
                else:
                    ivv = ivw[b, r, :]
                for j in range(dh // LANES):
                    sl = pl.ds(j * LANES, LANES)
                    rows[b, r, sl] = rows[b, r, sl] * ivv
                return 0

            lax.fori_loop(0, RB, srow, 0)
            out_descs[i] = pltpu.async_copy(
                rows.at[b], y_hbm.at[pl.ds(c_idx * N_PAD + rblk(i), RB)],
                ssem[b])
            if first:
                @pl.when(c_idx == 0)
                def _():
                    pltpu.sync_copy(ivw.at[b], inv_hbm.at[pl.ds(rblk(i), RB)])
        for i in sorted(out_descs):
            out_descs[i].wait()

    return pl.kernel(body, out_type=tuple(out_type) if first else out_type[0],
                     mesh=mesh, scratch_types=scratch,
                     compiler_params=pltpu.CompilerParams(
                         use_tc_tiling_on_sc=False))


TC_R = 2048


def _combine_call(dh_in, d_out, dh_out):
    xs_spec = pl.BlockSpec((2, TC_R, dh_in), lambda i: (0, i, 0))
    w_spec = pl.BlockSpec((2 * dh_in, d_out), lambda i: (0, 0))
    b_spec = pl.BlockSpec((1, d_out), lambda i: (0, 0))
    if dh_out:
        out_shape = jax.ShapeDtypeStruct((2, N_PAD, dh_out), f32)
        out_spec = pl.BlockSpec((2, TC_R, dh_out), lambda i: (0, i, 0))
    else:
        out_shape = jax.ShapeDtypeStruct((N_PAD, d_out), f32)
        out_spec = pl.BlockSpec((TC_R, d_out), lambda i: (i, 0))

    def body(x0r, s1r, s2r, war, wbr, wcr, br, out_r):
        h = (jnp.dot(x0r[0], war[:dh_in]) + jnp.dot(x0r[1], war[dh_in:])
             + jnp.dot(s1r[0], wbr[:dh_in]) + jnp.dot(s1r[1], wbr[dh_in:])
             + jnp.dot(s2r[0], wcr[:dh_in]) + jnp.dot(s2r[1], wcr[dh_in:]))
        h = jnp.maximum(h + br[0], 0.0)
        if dh_out:
            out_r[0] = h[:, :dh_out]
            rw = d_out - dh_out
            right = h[:, dh_out:d_out]
            if rw < dh_out:
                right = jnp.concatenate(
                    [right, jnp.zeros((TC_R, dh_out - rw), f32)], axis=1)
            out_r[1] = right
        else:
            out_r[...] = h

    return pl.pallas_call(
        body, out_shape=out_shape, grid=(N_PAD // TC_R,),
        in_specs=[xs_spec, xs_spec, xs_spec, w_spec, w_spec, w_spec, b_spec],
        out_specs=out_spec)


def _final_call():
    out_shape = jax.ShapeDtypeStruct((N_PAD, 10), f32)

    def body(h_r, w2_r, b2_r, w3_r, b3_r, out_r):
        t = jnp.dot(h_r[...], w2_r[...]) + b2_r[0]
        t = jnp.dot(t, w3_r[...]) + b3_r[0]
        t = t - jnp.max(t, axis=-1, keepdims=True)
        e = jnp.exp(t)
        out_r[...] = e / jnp.sum(e, axis=-1, keepdims=True)

    return pl.pallas_call(
        body, out_shape=out_shape, grid=(N_PAD // TC_R,),
        in_specs=[pl.BlockSpec((TC_R, 256), lambda i: (i, 0)),
                  pl.BlockSpec((256, 300), lambda i: (0, 0)),
                  pl.BlockSpec((1, 300), lambda i: (0, 0)),
                  pl.BlockSpec((300, 10), lambda i: (0, 0)),
                  pl.BlockSpec((1, 10), lambda i: (0, 0))],
        out_specs=pl.BlockSpec((TC_R, 10), lambda i: (i, 0)))


def _prep_w(W, din_real, din_p):
    W0, W1_, W2 = W[0::3], W[1::3], W[2::3]
    mats = (W0 - W2, W1_, 2.0 * W2)
    if din_p != din_real:
        mats = tuple(jnp.pad(m, ((0, din_p - din_real), (0, 0))) for m in mats)
    return mats


def kernel(x, edge_index, W1, b1, W2, b2, W3, b3, W4, b4, Wl2, bl2, Wl3, bl3):
    pad_e = E_PAD - E
    src = jnp.concatenate(
        [edge_index[0], jnp.full((pad_e,), N, jnp.int32)]
    ).reshape(NS, N_CHUNKS, C)
    dst = jnp.concatenate(
        [edge_index[1], jnp.full((pad_e,), N, jnp.int32)]
    ).reshape(NS, N_CHUNKS, C)
    xp = jnp.pad(x, ((0, N_PAD - N), (0, 0)))
    x0s = jnp.concatenate([xp[:, :64], xp[:, 64:]], axis=0)

    dims = [(128, 128, 48, 32), (48, 64, 96, 48),
            (96, 96, 128, 64), (128, 128, 256, 0)]
    Ws = [W1, W2, W3, W4]
    bs = [b1, b2, b3, b4]

    inv = None
    for li, (din, dinp, dout, dhout) in enumerate(dims):
        dh = dinp // 2
        x3 = x0s.reshape(2, N_PAD, dh)
        if li == 0:
            s1, inv = _spmm_call(dh, True)(src, dst, x0s)
        else:
            s1 = _spmm_call(dh, False)(src, dst, x0s, inv)
        s2 = _spmm_call(dh, False)(src, dst, s1, inv)
        wa, wb, wc = _prep_w(Ws[li], din, dinp)
        out = _combine_call(dh, dout, dhout)(
            x3, s1.reshape(2, N_PAD, dh), s2.reshape(2, N_PAD, dh),
            wa, wb, wc, bs[li].reshape(1, dout))
        x0s = out.reshape(2 * N_PAD, dhout) if dhout else out

    o = _final_call()(x0s, Wl2, bl2.reshape(1, 300), Wl3, bl3.reshape(1, 10))
    return o[:N]
